# Initial kernel scaffold; baseline (speedup 1.0000x reference)
import jax
import jax.numpy as jnp
from jax.experimental import pallas as pl


def kernel(x, tok_embed):
    raise NotImplementedError("write your pallas kernel here")


# speedup vs baseline: 1.8288x; 1.8288x over previous
import functools

import jax
import jax.numpy as jnp
from jax import lax
from jax.experimental import pallas as pl
from jax.experimental.pallas import tpu as pltpu
from jax.experimental.pallas import tpu_sc as plsc

EMBED = 64

_NC = 2
_NS = 16
_NW = _NC * _NS


@functools.lru_cache(maxsize=None)
def _build(B, D):
    rows_per_w = B // _NW
    R = rows_per_w // 128
    K = 4
    C = K * 128
    nchunk = rows_per_w // C

    mesh = plsc.VectorSubcoreMesh(core_axis_name="c", subcore_axis_name="s")

    @functools.partial(
        pl.kernel,
        mesh=mesh,
        out_type=jax.ShapeDtypeStruct((B, D), jnp.float32),
        scratch_types=[
            pltpu.VMEM((R, 128), jnp.int32),
            pltpu.VMEM((C, D), jnp.float32),
            pltpu.SemaphoreType.DMA,
        ],
        compiler_params=pltpu.CompilerParams(use_tc_tiling_on_sc=False),
    )
    def sc_gather(table_hbm, idx_hbm, out_hbm, idx_v, rows_v, sem):
        wid = lax.axis_index("s") * _NC + lax.axis_index("c")
        row0 = wid * R
        pltpu.sync_copy(idx_hbm.at[pl.ds(row0, R)], idx_v)

        def chunk(g, carry):
            descs = []
            for j in range(K):
                descs.append(pltpu.async_copy(
                    table_hbm.at[idx_v.at[g * K + j]],
                    rows_v.at[pl.ds(j * 128, 128)],
                    sem))
            for d in descs:
                d.wait()
            pltpu.sync_copy(rows_v, out_hbm.at[pl.ds((row0 + g * K) * 128, C)])
            return carry

        lax.fori_loop(0, nchunk, chunk, 0)

    return sc_gather


def kernel(x, tok_embed):
    s0, s1 = x.shape
    B = s0 * s1
    idx2d = x.reshape(B // 128, 128).astype(jnp.int32)
    out = _build(B, tok_embed.shape[1])(tok_embed, idx2d)
    return out.reshape(s0, s1, tok_embed.shape[1])
